# baseline (speedup 1.0000x reference)
import functools

import jax
import jax.numpy as jnp
from jax import lax
from jax.experimental import pallas as pl
from jax.experimental.pallas import tpu as pltpu
from jax.experimental.pallas import tpu_sc as plsc

_NC = 2
_NS = 16
_NW = _NC * _NS


@functools.lru_cache(maxsize=None)
def _make_gather(B, V, D, C):
    BW = B // _NW
    n_chunks = BW // C
    mesh = plsc.VectorSubcoreMesh(
        core_axis_name="c", subcore_axis_name="s",
        num_cores=_NC, num_subcores=_NS,
    )

    @functools.partial(
        pl.kernel,
        out_type=jax.ShapeDtypeStruct((B, D), jnp.float32),
        mesh=mesh,
        scratch_types=[
            pltpu.VMEM((C,), jnp.int32),
            pltpu.VMEM((C, D), jnp.float32),
            pltpu.SemaphoreType.DMA,
        ],
        compiler_params=pltpu.CompilerParams(use_tc_tiling_on_sc=False),
    )
    def gather_kernel(x_hbm, table_hbm, out_hbm, idx_v, rows_v, sem):
        wid = lax.axis_index("s") * _NC + lax.axis_index("c")
        base = wid * BW

        def body(i, carry):
            off = base + i * C
            pltpu.sync_copy(x_hbm.at[pl.ds(off, C)], idx_v)
            pltpu.async_copy(table_hbm.at[idx_v], rows_v, sem).wait()
            pltpu.sync_copy(rows_v, out_hbm.at[pl.ds(off, C)])
            return carry

        lax.fori_loop(0, n_chunks, body, 0)

    return gather_kernel


def kernel(x, table):
    B0, H = x.shape
    V, D = table.shape
    xf = x.reshape(-1).astype(jnp.int32)
    out = _make_gather(xf.size, V, D, 512)(xf, table)
    return out.reshape(B0, H, D)


# speedup vs baseline: 1.0530x; 1.0530x over previous
import functools

import jax
import jax.numpy as jnp
from jax import lax
from jax.experimental import pallas as pl
from jax.experimental.pallas import tpu as pltpu
from jax.experimental.pallas import tpu_sc as plsc

_NC = 2
_NS = 16
_NW = _NC * _NS


@functools.lru_cache(maxsize=None)
def _make_gather(B, V, D, C):
    BW = B // _NW
    n_chunks = BW // C
    assert n_chunks >= 2 and n_chunks % 2 == 0
    mesh = plsc.VectorSubcoreMesh(
        core_axis_name="c", subcore_axis_name="s",
        num_cores=_NC, num_subcores=_NS,
    )

    @functools.partial(
        pl.kernel,
        out_type=jax.ShapeDtypeStruct((B, D), jnp.float32),
        mesh=mesh,
        scratch_types=[
            [pltpu.VMEM((C,), jnp.int32)] * 2,
            [pltpu.VMEM((C, D), jnp.float32)] * 2,
            [pltpu.SemaphoreType.DMA] * 2,
            [pltpu.SemaphoreType.DMA] * 2,
        ],
        compiler_params=pltpu.CompilerParams(use_tc_tiling_on_sc=False),
    )
    def gather_kernel(x_hbm, table_hbm, out_hbm, idx_v, rows_v, g_sem, st_sem):
        wid = lax.axis_index("s") * _NC + lax.axis_index("c")
        base = wid * BW

        def chunk(i, b, first):
            off = base + i * C
            if not first:
                pltpu.make_async_copy(rows_v[b], out_hbm.at[pl.ds(0, C)],
                                      st_sem[b]).wait()
            pltpu.sync_copy(x_hbm.at[pl.ds(off, C)], idx_v[b])
            pltpu.async_copy(table_hbm.at[idx_v[b]], rows_v[b], g_sem[b]).wait()
            pltpu.async_copy(rows_v[b], out_hbm.at[pl.ds(off, C)], st_sem[b])

        for b in range(2):
            chunk(b, b, True)

        def body(j, carry):
            for b in range(2):
                chunk(2 * j + b, b, False)
            return carry

        lax.fori_loop(1, n_chunks // 2, body, 0)

        for b in range(2):
            pltpu.make_async_copy(rows_v[b], out_hbm.at[pl.ds(0, C)],
                                  st_sem[b]).wait()

    return gather_kernel


def kernel(x, table):
    B0, H = x.shape
    V, D = table.shape
    xf = x.reshape(-1).astype(jnp.int32)
    out = _make_gather(xf.size, V, D, 512)(xf, table)
    return out.reshape(B0, H, D)


# speedup vs baseline: 1.0632x; 1.0097x over previous
import functools

import jax
import jax.numpy as jnp
from jax import lax
from jax.experimental import pallas as pl
from jax.experimental.pallas import tpu as pltpu
from jax.experimental.pallas import tpu_sc as plsc

_NC = 2
_NS = 16
_NW = _NC * _NS


@functools.lru_cache(maxsize=None)
def _make_gather(B, V, D, C):
    BW = B // _NW
    n_chunks = BW // C
    assert n_chunks >= 2 and n_chunks % 2 == 0
    mesh = plsc.VectorSubcoreMesh(
        core_axis_name="c", subcore_axis_name="s",
        num_cores=_NC, num_subcores=_NS,
    )

    @functools.partial(
        pl.kernel,
        out_type=jax.ShapeDtypeStruct((B, D), jnp.float32),
        mesh=mesh,
        scratch_types=[
            [pltpu.VMEM((C,), jnp.int32)] * 2,
            [pltpu.VMEM((C, D), jnp.float32)] * 2,
            [pltpu.SemaphoreType.DMA] * 2,
            [pltpu.SemaphoreType.DMA] * 2,
        ],
        compiler_params=pltpu.CompilerParams(use_tc_tiling_on_sc=False),
    )
    def gather_kernel(x_hbm, table_hbm, out_hbm, idx_v, rows_v, g_sem, st_sem):
        wid = lax.axis_index("s") * _NC + lax.axis_index("c")
        base = wid * BW

        def load_idx(i, b):
            pltpu.sync_copy(x_hbm.at[pl.ds(base + i * C, C)], idx_v[b])

        def fire_gather(b):
            pltpu.async_copy(table_hbm.at[idx_v[b]], rows_v[b], g_sem[b])

        def wait_gather(b):
            pltpu.make_async_copy(table_hbm.at[idx_v[b]], rows_v[b],
                                  g_sem[b]).wait()

        def fire_store(i, b):
            pltpu.async_copy(rows_v[b], out_hbm.at[pl.ds(base + i * C, C)],
                             st_sem[b])

        def wait_store(b):
            pltpu.make_async_copy(rows_v[b], out_hbm.at[pl.ds(0, C)],
                                  st_sem[b]).wait()

        for b in range(2):
            load_idx(b, b)
            fire_gather(b)

        def body(j, carry):
            for b in range(2):
                i = 2 * j + b
                wait_gather(b)
                fire_store(i - 2, b)
                wait_store(b)
                load_idx(i, b)
                fire_gather(b)
            return carry

        lax.fori_loop(1, n_chunks // 2, body, 0)

        for b in range(2):
            i = n_chunks - 2 + b
            wait_gather(b)
            fire_store(i, b)
        for b in range(2):
            wait_store(b)

    return gather_kernel


def kernel(x, table):
    B0, H = x.shape
    V, D = table.shape
    xf = x.reshape(-1).astype(jnp.int32)
    out = _make_gather(xf.size, V, D, 800)(xf, table)
    return out.reshape(B0, H, D)


# speedup vs baseline: 1.0636x; 1.0004x over previous
import functools

import jax
import jax.numpy as jnp
from jax import lax
from jax.experimental import pallas as pl
from jax.experimental.pallas import tpu as pltpu
from jax.experimental.pallas import tpu_sc as plsc

_NC = 2
_NS = 16
_NW = _NC * _NS


@functools.lru_cache(maxsize=None)
def _make_gather(B, V, D, C):
    BW = B // _NW
    n_chunks = BW // C
    assert n_chunks >= 2 and n_chunks % 2 == 0
    mesh = plsc.VectorSubcoreMesh(
        core_axis_name="c", subcore_axis_name="s",
        num_cores=_NC, num_subcores=_NS,
    )

    @functools.partial(
        pl.kernel,
        out_type=jax.ShapeDtypeStruct((B, D), jnp.float32),
        mesh=mesh,
        scratch_types=[
            [pltpu.VMEM((C,), jnp.int32)] * 2,
            [pltpu.VMEM((C, D), jnp.float32)] * 2,
            [pltpu.SemaphoreType.DMA] * 2,
            [pltpu.SemaphoreType.DMA] * 2,
        ],
        compiler_params=pltpu.CompilerParams(use_tc_tiling_on_sc=False,
                                             skip_device_barrier=True),
    )
    def gather_kernel(x_hbm, table_hbm, out_hbm, idx_v, rows_v, g_sem, st_sem):
        wid = lax.axis_index("s") * _NC + lax.axis_index("c")
        base = wid * BW

        def load_idx(i, b):
            pltpu.sync_copy(x_hbm.at[pl.ds(base + i * C, C)], idx_v[b])

        def fire_gather(b):
            pltpu.async_copy(table_hbm.at[idx_v[b]], rows_v[b], g_sem[b])

        def wait_gather(b):
            pltpu.make_async_copy(table_hbm.at[idx_v[b]], rows_v[b],
                                  g_sem[b]).wait()

        def fire_store(i, b):
            pltpu.async_copy(rows_v[b], out_hbm.at[pl.ds(base + i * C, C)],
                             st_sem[b])

        def wait_store(b):
            pltpu.make_async_copy(rows_v[b], out_hbm.at[pl.ds(0, C)],
                                  st_sem[b]).wait()

        for b in range(2):
            load_idx(b, b)
            fire_gather(b)

        def body(j, carry):
            for b in range(2):
                i = 2 * j + b
                wait_gather(b)
                fire_store(i - 2, b)
                wait_store(b)
                load_idx(i, b)
                fire_gather(b)
            return carry

        lax.fori_loop(1, n_chunks // 2, body, 0)

        for b in range(2):
            i = n_chunks - 2 + b
            wait_gather(b)
            fire_store(i, b)
        for b in range(2):
            wait_store(b)

    return gather_kernel


def kernel(x, table):
    B0, H = x.shape
    V, D = table.shape
    xf = x.reshape(-1).astype(jnp.int32)
    out = _make_gather(xf.size, V, D, 800)(xf, table)
    return out.reshape(B0, H, D)
